# Initial kernel scaffold; baseline (speedup 1.0000x reference)
import jax
import jax.numpy as jnp
from jax.experimental import pallas as pl


def kernel(x, edge_index, batch, W1, b1, W2, b2, W3, b3, pc1w, pc1b, pc2w, pc2b, pc3w, pc3b, oq1w, oq1b, oq2w, oq2b, oq3w, oq3b, st1w, st1b, st2w, st2b, ot1w, ot1b, ot2w, ot2b, ot3w, ot3b):
    raise NotImplementedError("write your pallas kernel here")


# speedup vs baseline: 1.4679x; 1.4679x over previous
import functools

import jax
import jax.numpy as jnp
from jax import lax
from jax.experimental import pallas as pl
from jax.experimental.pallas import tpu as pltpu
from jax.experimental.pallas import tpu_sc as plsc

NC = 2
NS = 16
NW = NC * NS
LANES = 16

NPAD = 10240
RPW = NPAD // NW
CHUNK = 1024
GK = 128
PB = 64

NEG = -3.0e38
THRESH = -1.0e38


def _wid():
  return lax.axis_index("s") * NC + lax.axis_index("c")


def _mesh():
  return plsc.VectorSubcoreMesh(
      core_axis_name="c", subcore_axis_name="s", num_cores=NC, num_subcores=NS
  )


def _bucketize(src_pad, dst_pad, epad, capb):
  nchunk = epad // CHUNK
  buf = CHUNK + LANES

  def body(src_hbm, dst_hbm, bsrc, bdst, cnts, srcb, dstb, csrc, cdst, cbuf):
    wid = _wid()
    base = wid * RPW
    bbase = wid * capb
    sent_s = jnp.zeros((LANES,), jnp.int32)
    sent_d = jnp.full((LANES,), RPW, jnp.int32)

    def chunk_body(ci, total):
      pltpu.sync_copy(src_hbm.at[pl.ds(ci * CHUNK, CHUNK)], srcb)
      pltpu.sync_copy(dst_hbm.at[pl.ds(ci * CHUNK, CHUNK)], dstb)

      def vec_body(v, nfill):
        d = dstb[pl.ds(v * LANES, LANES)]
        s = srcb[pl.ds(v * LANES, LANES)]
        dl = d - base
        m = (dl >= 0) & (dl < RPW)
        cums = jnp.cumsum(m.astype(jnp.int32))
        lane = lax.iota(jnp.int32, LANES)
        pos = jnp.where(m, nfill + cums - 1, buf + lane)
        plsc.store_scatter(csrc, [pos], s)
        plsc.store_scatter(cdst, [pos], dl)
        return nfill + cums[LANES - 1]

      nfill = lax.fori_loop(0, CHUNK // LANES, vec_body, 0)
      spos = nfill + lax.iota(jnp.int32, LANES)
      plsc.store_scatter(csrc, [spos], sent_s)
      plsc.store_scatter(cdst, [spos], sent_d)
      off = pl.multiple_of(bbase + total, 8)
      pltpu.sync_copy(csrc.at[pl.ds(0, buf)], bsrc.at[pl.ds(off, buf)])
      pltpu.sync_copy(cdst.at[pl.ds(0, buf)], bdst.at[pl.ds(off, buf)])
      return total + ((nfill + 7) // 8) * 8

    total = lax.fori_loop(0, nchunk, chunk_body, 0)

    def fill_body(i, _):
      csrc[pl.ds(i * LANES, LANES)] = sent_s
      cdst[pl.ds(i * LANES, LANES)] = sent_d
      return 0

    lax.fori_loop(0, buf // LANES, fill_body, 0)
    off = pl.multiple_of(bbase + total, 8)
    pltpu.sync_copy(csrc.at[pl.ds(0, buf)], bsrc.at[pl.ds(off, buf)])
    pltpu.sync_copy(cdst.at[pl.ds(0, buf)], bdst.at[pl.ds(off, buf)])

    cbuf[...] = jnp.full((LANES,), 0, jnp.int32) + total
    pltpu.sync_copy(cbuf, cnts.at[wid])

  k = pl.kernel(
      body,
      out_type=[
          jax.ShapeDtypeStruct((NW * capb,), jnp.int32),
          jax.ShapeDtypeStruct((NW * capb,), jnp.int32),
          jax.ShapeDtypeStruct((NW, LANES), jnp.int32),
      ],
      mesh=_mesh(),
      compiler_params=pltpu.CompilerParams(needs_layout_passes=False),
      scratch_types=[
          pltpu.VMEM((CHUNK,), jnp.int32),
          pltpu.VMEM((CHUNK,), jnp.int32),
          pltpu.VMEM((buf + LANES,), jnp.int32),
          pltpu.VMEM((buf + LANES,), jnp.int32),
          pltpu.VMEM((LANES,), jnp.int32),
      ],
  )
  return k(src_pad, dst_pad)


def _seg_max_combine(bsrc, bdst, cnts, q, p, capb, relu):
  def body(bsrc_hbm, bdst_hbm, cnts_hbm, q_hbm, p_hbm, h_hbm,
           acc, rows, slist, dlist, cbuf, pbuf, sem):
    wid = _wid()
    base = wid * RPW
    bbase = wid * capb
    negv = jnp.full((LANES,), NEG, jnp.float32)

    def init_body(i, _):
      acc[i // 8, pl.ds((i % 8) * LANES, LANES)] = negv
      return 0

    lax.fori_loop(0, (RPW + 1) * 8, init_body, 0)

    pltpu.sync_copy(cnts_hbm.at[wid], cbuf)
    cnt = jnp.max(cbuf[...])
    nblk = (cnt + GK - 1) // GK

    def blk_body(g, _):
      off = pl.multiple_of(bbase + g * GK, 8)
      pltpu.sync_copy(bsrc_hbm.at[pl.ds(off, GK)], slist)
      pltpu.sync_copy(bdst_hbm.at[pl.ds(off, GK)], dlist)
      pltpu.async_copy(q_hbm.at[slist], rows, sem).wait()

      def edge_body(eg, _):
        dvec = dlist[pl.ds(eg * LANES, LANES)]
        for j in range(LANES):
          dl = dvec[j]
          e = eg * LANES + j
          for jc in range(8):
            sl = pl.ds(jc * LANES, LANES)
            acc[dl, sl] = jnp.maximum(acc[dl, sl], rows[e, sl])
        return 0

      lax.fori_loop(0, GK // LANES, edge_body, 0)
      return 0

    lax.fori_loop(0, nblk, blk_body, 0)

    def comb_body(pi, _):
      pltpu.sync_copy(p_hbm.at[pl.ds(base + pi * PB, PB)], pbuf)

      def row_body(i, _):
        r = pi * PB + i
        for j in range(8):
          sl = pl.ds(j * LANES, LANES)
          a = acc[r, sl]
          v = pbuf[i, sl] + a
          v = jnp.where(a > THRESH, v, 0.0)
          if relu:
            v = jnp.maximum(v, 0.0)
          pbuf[i, sl] = v
        return 0

      lax.fori_loop(0, PB, row_body, 0)
      pltpu.sync_copy(pbuf, h_hbm.at[pl.ds(base + pi * PB, PB)])
      return 0

    lax.fori_loop(0, RPW // PB, comb_body, 0)

  k = pl.kernel(
      body,
      out_type=jax.ShapeDtypeStruct((NPAD, 128), jnp.float32),
      mesh=_mesh(),
      compiler_params=pltpu.CompilerParams(needs_layout_passes=False),
      scratch_types=[
          pltpu.VMEM((RPW + 1, 128), jnp.float32),
          pltpu.VMEM((GK, 128), jnp.float32),
          pltpu.VMEM((GK,), jnp.int32),
          pltpu.VMEM((GK,), jnp.int32),
          pltpu.VMEM((LANES,), jnp.int32),
          pltpu.VMEM((PB, 128), jnp.float32),
          pltpu.SemaphoreType.DMA,
      ],
  )
  return k(bsrc, bdst, cnts, q, p)


def _pq_tc(h, a, b, bias):
  bm = 1024
  kdim = h.shape[1]

  def body(h_ref, a_ref, b_ref, bias_ref, p_ref, q_ref):
    hb = h_ref[...]
    p_ref[...] = jnp.dot(hb, a_ref[...], preferred_element_type=jnp.float32) + bias_ref[...]
    q_ref[...] = jnp.dot(hb, b_ref[...], preferred_element_type=jnp.float32)

  grid = (NPAD // bm,)
  return pl.pallas_call(
      body,
      grid=grid,
      in_specs=[
          pl.BlockSpec((bm, kdim), lambda i: (i, 0)),
          pl.BlockSpec((kdim, 128), lambda i: (0, 0)),
          pl.BlockSpec((kdim, 128), lambda i: (0, 0)),
          pl.BlockSpec((1, 128), lambda i: (0, 0)),
      ],
      out_specs=[
          pl.BlockSpec((bm, 128), lambda i: (i, 0)),
          pl.BlockSpec((bm, 128), lambda i: (i, 0)),
      ],
      out_shape=[
          jax.ShapeDtypeStruct((NPAD, 128), jnp.float32),
          jax.ShapeDtypeStruct((NPAD, 128), jnp.float32),
      ],
  )(h, a, b, bias)


def _pool_tc(batch3, h, g):
  bp = 512
  nb = NPAD // bp

  def body(b_ref, h_ref, gsum_ref, cnt_ref):
    @pl.when(pl.program_id(0) == 0)
    def _():
      gsum_ref[...] = jnp.zeros_like(gsum_ref)
      cnt_ref[...] = jnp.zeros_like(cnt_ref)

    bvec = b_ref[0, 0, :]
    onehot = (bvec[None, :] == lax.broadcasted_iota(jnp.int32, (g, bp), 0)
              ).astype(jnp.float32)
    gsum_ref[...] += jnp.dot(onehot, h_ref[...], preferred_element_type=jnp.float32)
    cnt_ref[...] += jnp.dot(onehot, jnp.ones((bp, 128), jnp.float32),
                            preferred_element_type=jnp.float32)

  return pl.pallas_call(
      body,
      grid=(nb,),
      in_specs=[
          pl.BlockSpec((1, 1, bp), lambda i: (i, 0, 0)),
          pl.BlockSpec((bp, 128), lambda i: (i, 0)),
      ],
      out_specs=[
          pl.BlockSpec((g, 128), lambda i: (0, 0)),
          pl.BlockSpec((g, 128), lambda i: (0, 0)),
      ],
      out_shape=[
          jax.ShapeDtypeStruct((g, 128), jnp.float32),
          jax.ShapeDtypeStruct((g, 128), jnp.float32),
      ],
  )(batch3, h)


def _heads_tc(gsum, cnt, ws, g):
  def body(gsum_ref, cnt_ref, *refs):
    w = [r[...] for r in refs[:22]]
    phys_ref, opt_ref, tox_ref, arom_ref = refs[22:]
    gv = gsum_ref[...] / jnp.maximum(cnt_ref[...], 1.0)

    def lin(x, i):
      return jnp.dot(x, w[2 * i], preferred_element_type=jnp.float32) + w[2 * i + 1]

    h = jax.nn.relu(lin(gv, 0))
    h = jax.nn.relu(lin(h, 1))
    phys_ref[...] = lin(h, 2)
    h = jax.nn.relu(lin(gv, 3))
    h = jax.nn.relu(lin(h, 4))
    opt_ref[...] = lin(h, 5)
    h = jax.nn.relu(lin(gv, 6))
    tox_ref[...] = lin(h, 7)
    h = jax.nn.relu(lin(gv, 8))
    h = jax.nn.relu(lin(h, 9))
    arom_ref[...] = lin(h, 10)

  in_specs = [pl.BlockSpec((g, 128), lambda: (0, 0)),
              pl.BlockSpec((g, 128), lambda: (0, 0))]
  for wmat in ws:
    in_specs.append(pl.BlockSpec(wmat.shape, lambda: (0,) * wmat.ndim))
  return pl.pallas_call(
      body,
      in_specs=in_specs,
      out_specs=[pl.BlockSpec((g, 128), lambda: (0, 0))] * 4,
      out_shape=[jax.ShapeDtypeStruct((g, 128), jnp.float32)] * 4,
  )(gsum, cnt, *ws)


def _pad_w(w):
  r, c = w.shape
  return jnp.pad(w, ((0, 128 - r), (0, 128 - c)))


def _pad_b(b):
  return jnp.pad(b, (0, 128 - b.shape[0])).reshape(1, 128)


def kernel(x, edge_index, batch, W1, b1, W2, b2, W3, b3, pc1w, pc1b, pc2w,
           pc2b, pc3w, pc3b, oq1w, oq1b, oq2w, oq2b, oq3w, oq3b, st1w, st1b,
           st2w, st2b, ot1w, ot1b, ot2w, ot2b, ot3w, ot3b):
  n, fin = x.shape
  e = edge_index.shape[1]
  g = 64

  epad = ((e + CHUNK - 1) // CHUNK) * CHUNK
  capb = epad + 2 * CHUNK
  src_pad = jnp.concatenate(
      [edge_index[0], jnp.zeros((epad - e,), jnp.int32)])
  dst_pad = jnp.concatenate(
      [edge_index[1], jnp.full((epad - e,), jnp.int32(2**31 - 1))])

  bsrc, bdst, cnts = _bucketize(src_pad, dst_pad, epad, capb)

  kin = 16
  x_pad = jnp.pad(x, ((0, NPAD - n), (0, kin - fin)))

  def split(w, f):
    a = w[:f] - w[f:]
    return a, w[f:]

  a1, bb1 = split(W1, fin)
  a1 = jnp.pad(a1, ((0, kin - fin), (0, 0)))
  bb1 = jnp.pad(bb1, ((0, kin - fin), (0, 0)))
  a2, bb2 = split(W2, 128)
  a3, bb3 = split(W3, 128)

  p1, q1 = _pq_tc(x_pad, a1, bb1, b1.reshape(1, 128))
  h1 = _seg_max_combine(bsrc, bdst, cnts, q1, p1, capb, relu=True)
  p2, q2 = _pq_tc(h1, a2, bb2, b2.reshape(1, 128))
  h2 = _seg_max_combine(bsrc, bdst, cnts, q2, p2, capb, relu=True)
  p3, q3 = _pq_tc(h2, a3, bb3, b3.reshape(1, 128))
  h3 = _seg_max_combine(bsrc, bdst, cnts, q3, p3, capb, relu=False)

  batch3 = jnp.pad(batch, (0, NPAD - n), constant_values=g).reshape(
      NPAD // 512, 1, 512)
  gsum, cnt = _pool_tc(batch3, h3, g)

  ws = [_pad_w(pc1w), _pad_b(pc1b), _pad_w(pc2w), _pad_b(pc2b),
        _pad_w(pc3w), _pad_b(pc3b), _pad_w(oq1w), _pad_b(oq1b),
        _pad_w(oq2w), _pad_b(oq2b), _pad_w(oq3w), _pad_b(oq3b),
        _pad_w(st1w), _pad_b(st1b), _pad_w(st2w), _pad_b(st2b),
        _pad_w(ot1w), _pad_b(ot1b), _pad_w(ot2w), _pad_b(ot2b),
        _pad_w(ot3w), _pad_b(ot3b)]
  phys, opt, tox, arom = _heads_tc(gsum, cnt, ws, g)
  return (phys[:, :4], opt[:, :3], tox[:, :2], arom[:, :1])


# speedup vs baseline: 1.5242x; 1.0383x over previous
import functools

import jax
import jax.numpy as jnp
from jax import lax
from jax.experimental import pallas as pl
from jax.experimental.pallas import tpu as pltpu
from jax.experimental.pallas import tpu_sc as plsc

NC = 2
NS = 16
NW = NC * NS
LANES = 16

NPAD = 10240
RPW = NPAD // NW
CHUNK = 1024
GK = 128
PB = 64

NEG = -3.0e38
THRESH = -1.0e38


def _wid():
  return lax.axis_index("s") * NC + lax.axis_index("c")


def _mesh():
  return plsc.VectorSubcoreMesh(
      core_axis_name="c", subcore_axis_name="s", num_cores=NC, num_subcores=NS
  )


def _bucketize(src_pad, dst_pad, epad, capb):
  nchunk = epad // CHUNK
  buf = CHUNK + LANES

  def body(src_hbm, dst_hbm, bsrc, bdst, cnts, srcb, dstb, csrc, cdst, cbuf):
    wid = _wid()
    base = wid * RPW
    bbase = wid * capb
    sent_s = jnp.zeros((LANES,), jnp.int32)
    sent_d = jnp.full((LANES,), RPW, jnp.int32)

    def chunk_body(ci, total):
      pltpu.sync_copy(src_hbm.at[pl.ds(ci * CHUNK, CHUNK)], srcb)
      pltpu.sync_copy(dst_hbm.at[pl.ds(ci * CHUNK, CHUNK)], dstb)

      def vec_body(v, nfill):
        d = dstb[pl.ds(v * LANES, LANES)]
        s = srcb[pl.ds(v * LANES, LANES)]
        dl = d - base
        m = (dl >= 0) & (dl < RPW)
        cums = jnp.cumsum(m.astype(jnp.int32))
        lane = lax.iota(jnp.int32, LANES)
        pos = jnp.where(m, nfill + cums - 1, buf + lane)
        plsc.store_scatter(csrc, [pos], s)
        plsc.store_scatter(cdst, [pos], dl)
        return nfill + cums[LANES - 1]

      nfill = lax.fori_loop(0, CHUNK // LANES, vec_body, 0)
      spos = nfill + lax.iota(jnp.int32, LANES)
      plsc.store_scatter(csrc, [spos], sent_s)
      plsc.store_scatter(cdst, [spos], sent_d)
      off = pl.multiple_of(bbase + total, 8)
      pltpu.sync_copy(csrc.at[pl.ds(0, buf)], bsrc.at[pl.ds(off, buf)])
      pltpu.sync_copy(cdst.at[pl.ds(0, buf)], bdst.at[pl.ds(off, buf)])
      return total + ((nfill + 7) // 8) * 8

    total = lax.fori_loop(0, nchunk, chunk_body, 0)

    def fill_body(i, _):
      csrc[pl.ds(i * LANES, LANES)] = sent_s
      cdst[pl.ds(i * LANES, LANES)] = sent_d
      return 0

    lax.fori_loop(0, buf // LANES, fill_body, 0)
    off = pl.multiple_of(bbase + total, 8)
    pltpu.sync_copy(csrc.at[pl.ds(0, buf)], bsrc.at[pl.ds(off, buf)])
    pltpu.sync_copy(cdst.at[pl.ds(0, buf)], bdst.at[pl.ds(off, buf)])

    cbuf[...] = jnp.full((LANES,), 0, jnp.int32) + total
    pltpu.sync_copy(cbuf, cnts.at[wid])

  k = pl.kernel(
      body,
      out_type=[
          jax.ShapeDtypeStruct((NW * capb,), jnp.int32),
          jax.ShapeDtypeStruct((NW * capb,), jnp.int32),
          jax.ShapeDtypeStruct((NW, LANES), jnp.int32),
      ],
      mesh=_mesh(),
      compiler_params=pltpu.CompilerParams(needs_layout_passes=False),
      scratch_types=[
          pltpu.VMEM((CHUNK,), jnp.int32),
          pltpu.VMEM((CHUNK,), jnp.int32),
          pltpu.VMEM((buf + LANES,), jnp.int32),
          pltpu.VMEM((buf + LANES,), jnp.int32),
          pltpu.VMEM((LANES,), jnp.int32),
      ],
  )
  return k(src_pad, dst_pad)


def _seg_max_combine(bsrc, bdst, cnts, q, p, capb, relu):
  def body(bsrc_hbm, bdst_hbm, cnts_hbm, q_hbm, p_hbm, h_hbm,
           acc, rows, slist, dlist, cbuf, pbuf, sem):
    wid = _wid()
    base = wid * RPW
    bbase = wid * capb
    negv = jnp.full((LANES,), NEG, jnp.float32)

    def init_body(i, _):
      acc[i // 8, pl.ds((i % 8) * LANES, LANES)] = negv
      return 0

    lax.fori_loop(0, (RPW + 1) * 8, init_body, 0)

    pltpu.sync_copy(cnts_hbm.at[wid], cbuf)
    cnt = jnp.max(cbuf[...])
    nblk = (cnt + GK - 1) // GK

    def fetch(g, buf):
      off = pl.multiple_of(bbase + g * GK, 8)
      pltpu.sync_copy(bsrc_hbm.at[pl.ds(off, GK)], slist.at[buf])
      pltpu.sync_copy(bdst_hbm.at[pl.ds(off, GK)], dlist.at[buf])
      pltpu.async_copy(q_hbm.at[slist.at[buf]], rows.at[buf], sem.at[buf])

    def wait_for(buf):
      pltpu.make_async_copy(
          q_hbm.at[slist.at[buf]], rows.at[buf], sem.at[buf]).wait()

    def drain(buf):
      def edge_body(eg, _):
        dvec = dlist[buf, pl.ds(eg * LANES, LANES)]
        for j in range(LANES):
          dl = dvec[j]
          e = eg * LANES + j
          av = [acc[dl, pl.ds(jc * LANES, LANES)] for jc in range(8)]
          rv = [rows[buf, e, pl.ds(jc * LANES, LANES)] for jc in range(8)]
          mx = [jnp.maximum(a, r) for a, r in zip(av, rv)]
          for jc in range(8):
            acc[dl, pl.ds(jc * LANES, LANES)] = mx[jc]
        return 0

      lax.fori_loop(0, GK // LANES, edge_body, 0)

    @pl.when(nblk > 0)
    def _():
      fetch(0, 0)

    def blk_body(g, _):
      b = lax.rem(g, 2)

      @pl.when(g + 1 < nblk)
      def _():
        fetch(g + 1, lax.rem(g + 1, 2))

      wait_for(b)
      drain(b)
      return 0

    lax.fori_loop(0, nblk, blk_body, 0)

    def comb_body(pi, _):
      pltpu.sync_copy(p_hbm.at[pl.ds(base + pi * PB, PB)], pbuf)

      def row_body(i, _):
        r = pi * PB + i
        for j in range(8):
          sl = pl.ds(j * LANES, LANES)
          a = acc[r, sl]
          v = pbuf[i, sl] + a
          v = jnp.where(a > THRESH, v, 0.0)
          if relu:
            v = jnp.maximum(v, 0.0)
          pbuf[i, sl] = v
        return 0

      lax.fori_loop(0, PB, row_body, 0)
      pltpu.sync_copy(pbuf, h_hbm.at[pl.ds(base + pi * PB, PB)])
      return 0

    lax.fori_loop(0, RPW // PB, comb_body, 0)

  k = pl.kernel(
      body,
      out_type=jax.ShapeDtypeStruct((NPAD, 128), jnp.float32),
      mesh=_mesh(),
      compiler_params=pltpu.CompilerParams(needs_layout_passes=False),
      scratch_types=[
          pltpu.VMEM((RPW + 1, 128), jnp.float32),
          pltpu.VMEM((2, GK, 128), jnp.float32),
          pltpu.VMEM((2, GK), jnp.int32),
          pltpu.VMEM((2, GK), jnp.int32),
          pltpu.VMEM((LANES,), jnp.int32),
          pltpu.VMEM((PB, 128), jnp.float32),
          pltpu.SemaphoreType.DMA((2,)),
      ],
  )
  return k(bsrc, bdst, cnts, q, p)


def _pq_tc(h, a, b, bias):
  bm = 1024
  kdim = h.shape[1]

  def body(h_ref, a_ref, b_ref, bias_ref, p_ref, q_ref):
    hb = h_ref[...]
    p_ref[...] = jnp.dot(hb, a_ref[...], preferred_element_type=jnp.float32, precision=lax.Precision.HIGHEST) + bias_ref[...]
    q_ref[...] = jnp.dot(hb, b_ref[...], preferred_element_type=jnp.float32, precision=lax.Precision.HIGHEST)

  grid = (NPAD // bm,)
  return pl.pallas_call(
      body,
      grid=grid,
      in_specs=[
          pl.BlockSpec((bm, kdim), lambda i: (i, 0)),
          pl.BlockSpec((kdim, 128), lambda i: (0, 0)),
          pl.BlockSpec((kdim, 128), lambda i: (0, 0)),
          pl.BlockSpec((1, 128), lambda i: (0, 0)),
      ],
      out_specs=[
          pl.BlockSpec((bm, 128), lambda i: (i, 0)),
          pl.BlockSpec((bm, 128), lambda i: (i, 0)),
      ],
      out_shape=[
          jax.ShapeDtypeStruct((NPAD, 128), jnp.float32),
          jax.ShapeDtypeStruct((NPAD, 128), jnp.float32),
      ],
  )(h, a, b, bias)


def _pool_tc(batch3, h, g):
  bp = 512
  nb = NPAD // bp

  def body(b_ref, h_ref, gsum_ref, cnt_ref):
    @pl.when(pl.program_id(0) == 0)
    def _():
      gsum_ref[...] = jnp.zeros_like(gsum_ref)
      cnt_ref[...] = jnp.zeros_like(cnt_ref)

    bvec = b_ref[0, 0, :]
    onehot = (bvec[None, :] == lax.broadcasted_iota(jnp.int32, (g, bp), 0)
              ).astype(jnp.float32)
    gsum_ref[...] += jnp.dot(onehot, h_ref[...], preferred_element_type=jnp.float32, precision=lax.Precision.HIGHEST)
    cnt_ref[...] += jnp.dot(onehot, jnp.ones((bp, 128), jnp.float32),
                            preferred_element_type=jnp.float32, precision=lax.Precision.HIGHEST)

  return pl.pallas_call(
      body,
      grid=(nb,),
      in_specs=[
          pl.BlockSpec((1, 1, bp), lambda i: (i, 0, 0)),
          pl.BlockSpec((bp, 128), lambda i: (i, 0)),
      ],
      out_specs=[
          pl.BlockSpec((g, 128), lambda i: (0, 0)),
          pl.BlockSpec((g, 128), lambda i: (0, 0)),
      ],
      out_shape=[
          jax.ShapeDtypeStruct((g, 128), jnp.float32),
          jax.ShapeDtypeStruct((g, 128), jnp.float32),
      ],
  )(batch3, h)


def _heads_tc(gsum, cnt, ws, g):
  def body(gsum_ref, cnt_ref, *refs):
    w = [r[...] for r in refs[:22]]
    phys_ref, opt_ref, tox_ref, arom_ref = refs[22:]
    gv = gsum_ref[...] / jnp.maximum(cnt_ref[...], 1.0)

    def lin(x, i):
      return jnp.dot(x, w[2 * i], preferred_element_type=jnp.float32) + w[2 * i + 1]

    h = jax.nn.relu(lin(gv, 0))
    h = jax.nn.relu(lin(h, 1))
    phys_ref[...] = lin(h, 2)
    h = jax.nn.relu(lin(gv, 3))
    h = jax.nn.relu(lin(h, 4))
    opt_ref[...] = lin(h, 5)
    h = jax.nn.relu(lin(gv, 6))
    tox_ref[...] = lin(h, 7)
    h = jax.nn.relu(lin(gv, 8))
    h = jax.nn.relu(lin(h, 9))
    arom_ref[...] = lin(h, 10)

  in_specs = [pl.BlockSpec((g, 128), lambda: (0, 0)),
              pl.BlockSpec((g, 128), lambda: (0, 0))]
  for wmat in ws:
    in_specs.append(pl.BlockSpec(wmat.shape, lambda: (0,) * wmat.ndim))
  return pl.pallas_call(
      body,
      in_specs=in_specs,
      out_specs=[pl.BlockSpec((g, 128), lambda: (0, 0))] * 4,
      out_shape=[jax.ShapeDtypeStruct((g, 128), jnp.float32)] * 4,
  )(gsum, cnt, *ws)


def _pad_w(w):
  r, c = w.shape
  return jnp.pad(w, ((0, 128 - r), (0, 128 - c)))


def _pad_b(b):
  return jnp.pad(b, (0, 128 - b.shape[0])).reshape(1, 128)


def kernel(x, edge_index, batch, W1, b1, W2, b2, W3, b3, pc1w, pc1b, pc2w,
           pc2b, pc3w, pc3b, oq1w, oq1b, oq2w, oq2b, oq3w, oq3b, st1w, st1b,
           st2w, st2b, ot1w, ot1b, ot2w, ot2b, ot3w, ot3b):
  n, fin = x.shape
  e = edge_index.shape[1]
  g = 64

  epad = ((e + CHUNK - 1) // CHUNK) * CHUNK
  capb = epad + 2 * CHUNK
  src_pad = jnp.concatenate(
      [edge_index[0], jnp.zeros((epad - e,), jnp.int32)])
  dst_pad = jnp.concatenate(
      [edge_index[1], jnp.full((epad - e,), jnp.int32(2**31 - 1))])

  bsrc, bdst, cnts = _bucketize(src_pad, dst_pad, epad, capb)

  kin = 16
  x_pad = jnp.pad(x, ((0, NPAD - n), (0, kin - fin)))

  def split(w, f):
    a = w[:f] - w[f:]
    return a, w[f:]

  a1, bb1 = split(W1, fin)
  a1 = jnp.pad(a1, ((0, kin - fin), (0, 0)))
  bb1 = jnp.pad(bb1, ((0, kin - fin), (0, 0)))
  a2, bb2 = split(W2, 128)
  a3, bb3 = split(W3, 128)

  p1, q1 = _pq_tc(x_pad, a1, bb1, b1.reshape(1, 128))
  h1 = _seg_max_combine(bsrc, bdst, cnts, q1, p1, capb, relu=True)
  p2, q2 = _pq_tc(h1, a2, bb2, b2.reshape(1, 128))
  h2 = _seg_max_combine(bsrc, bdst, cnts, q2, p2, capb, relu=True)
  p3, q3 = _pq_tc(h2, a3, bb3, b3.reshape(1, 128))
  h3 = _seg_max_combine(bsrc, bdst, cnts, q3, p3, capb, relu=False)

  batch3 = jnp.pad(batch, (0, NPAD - n), constant_values=g).reshape(
      NPAD // 512, 1, 512)
  gsum, cnt = _pool_tc(batch3, h3, g)

  ws = [_pad_w(pc1w), _pad_b(pc1b), _pad_w(pc2w), _pad_b(pc2b),
        _pad_w(pc3w), _pad_b(pc3b), _pad_w(oq1w), _pad_b(oq1b),
        _pad_w(oq2w), _pad_b(oq2b), _pad_w(oq3w), _pad_b(oq3b),
        _pad_w(st1w), _pad_b(st1b), _pad_w(st2w), _pad_b(st2b),
        _pad_w(ot1w), _pad_b(ot1b), _pad_w(ot2w), _pad_b(ot2b),
        _pad_w(ot3w), _pad_b(ot3b)]
  phys, opt, tox, arom = _heads_tc(gsum, cnt, ws, g)
  return (phys[:, :4], opt[:, :3], tox[:, :2], arom[:, :1])
